# Initial kernel scaffold; baseline (speedup 1.0000x reference)
import functools

import jax
import jax.numpy as jnp
from jax import lax
from jax.experimental import pallas as pl
from jax.experimental.pallas import tpu as pltpu
from jax.experimental.pallas import tpu_sc as plsc

L = 16


def _scalarize(v):
    return v if v.ndim == 0 else jnp.max(v)


def _make_body(n, wn, nw, chunk, k_lo, k_hi):
    per_w = n // nw
    nch = per_w // chunk
    wper = wn // nw
    wch = wper // chunk
    assert per_w % chunk == 0 and wper % chunk == 0

    out_type = jax.ShapeDtypeStruct((L,), jnp.float32)
    scratch_types = [
        pltpu.VMEM((2 * chunk,), jnp.float32),
        pltpu.VMEM((8192,), jnp.int32),
        pltpu.VMEM((512,), jnp.int32),
        pltpu.VMEM((512,), jnp.int32),
        pltpu.VMEM((512,), jnp.int32),
        pltpu.VMEM((32,), jnp.float32),
        pltpu.VMEM((32,), jnp.float32),
        pltpu.VMEM((L,), jnp.float32),
        pltpu.VMEM_SHARED((nw * 512,), jnp.int32),
        pltpu.VMEM_SHARED((nw * 32,), jnp.float32),
        pltpu.SemaphoreType.DMA,
    ]

    def body(x_hbm, w_hbm, out_hbm, buf, hist, folded, gh, tmp,
             wacc, wtmp, outv, stage_sh, wstage_sh, sem):
        wid = lax.axis_index("s")
        iota = lax.iota(jnp.int32, L)
        one_i = jnp.full((L,), 1, jnp.int32)
        c255 = jnp.full((L,), 255, jnp.int32)
        c31 = jnp.full((L,), 31, jnp.int32)
        minint = jnp.full((L,), -(2 ** 31), jnp.int32)
        maxint = jnp.full((L,), 2 ** 31 - 1, jnp.int32)

        def stream(hbm_ref, base, num_chunks, compute_fn):
            pltpu.make_async_copy(
                hbm_ref.at[pl.ds(base, chunk)],
                buf.at[pl.ds(0, chunk)], sem).start()

            @pl.loop(0, num_chunks)
            def _(g):
                sel = lax.rem(g, 2)

                @pl.when(g + 1 < num_chunks)
                def _():
                    pltpu.make_async_copy(
                        hbm_ref.at[pl.ds(base + (g + 1) * chunk, chunk)],
                        buf.at[pl.ds((1 - sel) * chunk, chunk)], sem).start()

                pltpu.make_async_copy(
                    hbm_ref.at[pl.ds(base + g * chunk, chunk)],
                    buf.at[pl.ds(sel * chunk, chunk)], sem).wait()
                compute_fn(sel)

        def keys_of(a):
            b = plsc.bitcast(a, jnp.int32)
            m = lax.shift_right_arithmetic(b, c31)
            return lax.bitwise_xor(b, lax.bitwise_or(m, minint))

        def wcompute(sel):
            sa0 = wacc[pl.ds(0, L)]
            sq0 = wacc[pl.ds(L, L)]

            @pl.loop(0, chunk // L, init_carry=(sa0, sq0))
            def acc(i, c):
                sa, sq = c
                a = buf[pl.ds(sel * chunk + i * L, L)]
                return (sa + jnp.abs(a), sq + a * a)

            sa, sq = acc
            wacc[pl.ds(0, L)] = sa
            wacc[pl.ds(L, L)] = sq

        wacc[pl.ds(0, L)] = jnp.zeros((L,), jnp.float32)
        wacc[pl.ds(L, L)] = jnp.zeros((L,), jnp.float32)
        stream(w_hbm, wid * wper, wch, wcompute)
        plsc.subcore_barrier()
        pltpu.sync_copy(wacc, wstage_sh.at[pl.ds(wid * 32, 32)])
        plsc.subcore_barrier()

        def select(row_base, thresh):
            def bod(j, c):
                r, jstar, rbef = c
                s = jnp.sum(gh[pl.ds(row_base + j * L, L)])
                newr = r + s
                crossed = jnp.logical_and(r < thresh, newr >= thresh)
                jstar = jnp.where(crossed, j, jstar)
                rbef = jnp.where(crossed, r, rbef)
                return (newr, jstar, rbef)

            _, jstar, rbef = lax.fori_loop(
                0, 16, bod,
                (jnp.int32(0), jnp.int32(0), jnp.int32(0)))
            v = gh[pl.ds(row_base + jstar * L, L)]
            cum = plsc.cumsum(v) + rbef
            off = _scalarize(plsc.all_reduce_ffs(cum >= thresh))
            byte = jstar * L + off
            hv = jnp.sum(jnp.where(iota == off, v, 0))
            cumat = jnp.sum(jnp.where(iota == off, cum, 0))
            return byte, cumat - hv

        def radix_pass(p, plo, phi, cblo, cbhi):
            sh_byte = jnp.full((L,), 24 - 8 * p, jnp.int32)
            sh_hi = jnp.full((L,), max(32 - 8 * p, 0), jnp.int32)
            lane_base = iota * 256

            @pl.loop(0, 8192 // L)
            def _(i):
                hist[pl.ds(i * L, L)] = jnp.zeros((L,), jnp.int32)

            def compute_fn(sel):
                @pl.loop(0, chunk // L)
                def _(i):
                    a = buf[pl.ds(sel * chunk + i * L, L)]
                    key = keys_of(a)
                    byte = lax.bitwise_and(
                        lax.shift_right_logical(key, sh_byte), c255)
                    idx = byte + lane_base
                    if p == 0:
                        plsc.addupdate_scatter(hist, [idx], one_i)
                    else:
                        hi = lax.shift_right_logical(key, sh_hi)
                        plsc.addupdate_scatter(
                            hist, [idx], one_i, mask=(hi == plo))
                        plsc.addupdate_scatter(
                            hist, [idx + 4096], one_i, mask=(hi == phi))

            stream(x_hbm, wid * per_w, nch, compute_fn)

            ntarget = 1 if p == 0 else 2

            @pl.loop(0, ntarget * 16)
            def _(tj):
                t = tj // 16
                j = lax.rem(tj, 16)

                @pl.loop(0, 16, init_carry=jnp.zeros((L,), jnp.int32))
                def acc(l, c):
                    return c + hist[pl.ds(t * 4096 + l * 256 + j * L, L)]

                folded[pl.ds(t * 256 + j * L, L)] = acc

            plsc.subcore_barrier()
            pltpu.sync_copy(folded, stage_sh.at[pl.ds(wid * 512, 512)])
            plsc.subcore_barrier()

            @pl.loop(0, 512 // L)
            def _(i):
                gh[pl.ds(i * L, L)] = jnp.zeros((L,), jnp.int32)

            @pl.loop(0, nw)
            def _(w):
                pltpu.sync_copy(stage_sh.at[pl.ds(w * 512, 512)], tmp)

                @pl.loop(0, 512 // L)
                def _(i):
                    gh[pl.ds(i * L, L)] = (
                        gh[pl.ds(i * L, L)] + tmp[pl.ds(i * L, L)])

            byte_lo, below_lo = select(0, k_lo - cblo)
            byte_hi, below_hi = select(0 if p == 0 else 256, k_hi - cbhi)
            plo = lax.bitwise_or(lax.shift_left(plo, jnp.int32(8)), byte_lo)
            phi = lax.bitwise_or(lax.shift_left(phi, jnp.int32(8)), byte_hi)
            return plo, phi, cblo + below_lo, cbhi + below_hi

        plo = phi = cblo = cbhi = jnp.int32(0)
        for p in range(4):
            plo, phi, cblo, cbhi = radix_pass(p, plo, phi, cblo, cbhi)

        z16 = jnp.zeros((L,), jnp.float32)

        @pl.loop(0, nw, init_carry=(z16, z16))
        def wtot(w, c):
            sa, sq = c
            pltpu.sync_copy(wstage_sh.at[pl.ds(w * 32, 32)], wtmp)
            return (sa + wtmp[pl.ds(0, L)], sq + wtmp[pl.ds(L, L)])

        sab = jnp.sum(wtot[0])
        ssq = jnp.sum(wtot[1])

        kv = jnp.where(iota == 0, phi, jnp.where(iota == 1, plo, 0))
        s = lax.shift_right_arithmetic(kv, c31)
        msk = lax.bitwise_xor(
            minint, lax.bitwise_and(maxint, lax.bitwise_not(s)))
        vals = plsc.bitcast(lax.bitwise_xor(kv, msk), jnp.float32)
        res = jnp.where(iota <= 1, vals, 0.0)
        res = jnp.where(iota == 2, sab, res)
        res = jnp.where(iota == 3, ssq, res)
        outv[...] = res

        @pl.when(wid == 0)
        def _():
            pltpu.sync_copy(outv, out_hbm)

    return body, out_type, scratch_types


@functools.lru_cache(maxsize=None)
def _build(n, wn):
    nw = 16
    chunk = 32768
    per_low = 0.1 * 0.01
    per_high = 99.9 * 0.01
    k_lo = max(int(per_low * n), 1)
    k_hi = int(per_high * n)
    body, out_type, scratch_types = _make_body(n, wn, nw, chunk, k_lo, k_hi)
    mesh = plsc.VectorSubcoreMesh(
        core_axis_name="c", subcore_axis_name="s", num_cores=1)
    return pl.kernel(body, out_type=out_type, mesh=mesh,
                     scratch_types=scratch_types)


def kernel(x, weight):
    xf = x.reshape(-1)
    wf = weight.reshape(-1)
    fn = _build(xf.shape[0], wf.shape[0])
    res = fn(xf, wf)
    upper = res[0]
    lower = res[1]
    w_abs_mean = res[2] / wf.shape[0]
    w_std_sawb = jnp.sqrt(res[3] / wf.shape[0])
    w_clip = -12.8 * w_abs_mean + 12.68 * w_std_sawb
    return jnp.stack([upper, lower, w_clip])


# speedup vs baseline: 8.8971x; 8.8971x over previous
import functools

import jax
import jax.numpy as jnp
from jax import lax
from jax.experimental import pallas as pl
from jax.experimental.pallas import tpu as pltpu
from jax.experimental.pallas import tpu_sc as plsc

L = 16


def _scalarize(v):
    return v if v.ndim == 0 else jnp.max(v)


def _make_body(n, wn, nw, chunk, k_lo, k_hi):
    per_w = n // nw
    nch = per_w // chunk
    wper = wn // nw
    wch = wper // chunk
    assert per_w % chunk == 0 and wper % chunk == 0

    out_type = jax.ShapeDtypeStruct((L,), jnp.float32)
    scratch_types = [
        pltpu.VMEM((2 * chunk,), jnp.float32),
        pltpu.VMEM((8192,), jnp.int32),
        pltpu.VMEM((512,), jnp.int32),
        pltpu.VMEM((512,), jnp.int32),
        pltpu.VMEM((512,), jnp.int32),
        pltpu.VMEM((32,), jnp.float32),
        pltpu.VMEM((32,), jnp.float32),
        pltpu.VMEM((L,), jnp.float32),
        pltpu.VMEM_SHARED((nw * 512,), jnp.int32),
        pltpu.VMEM_SHARED((nw * 32,), jnp.float32),
        pltpu.SemaphoreType.DMA,
    ]

    def body(x_hbm, w_hbm, out_hbm, buf, hist, folded, gh, tmp,
             wacc, wtmp, outv, stage_sh, wstage_sh, sem):
        wid = lax.axis_index("s")
        iota = lax.iota(jnp.int32, L)
        one_i = jnp.full((L,), 1, jnp.int32)
        c255 = jnp.full((L,), 255, jnp.int32)
        c31 = jnp.full((L,), 31, jnp.int32)
        minint = jnp.full((L,), -(2 ** 31), jnp.int32)
        maxint = jnp.full((L,), 2 ** 31 - 1, jnp.int32)

        def stream(hbm_ref, base, num_chunks, compute_fn):
            pltpu.make_async_copy(
                hbm_ref.at[pl.ds(base, chunk)],
                buf.at[pl.ds(0, chunk)], sem).start()

            @pl.loop(0, num_chunks)
            def _(g):
                sel = lax.rem(g, 2)

                @pl.when(g + 1 < num_chunks)
                def _():
                    pltpu.make_async_copy(
                        hbm_ref.at[pl.ds(base + (g + 1) * chunk, chunk)],
                        buf.at[pl.ds((1 - sel) * chunk, chunk)], sem).start()

                pltpu.make_async_copy(
                    hbm_ref.at[pl.ds(base + g * chunk, chunk)],
                    buf.at[pl.ds(sel * chunk, chunk)], sem).wait()
                compute_fn(sel)

        def keys_of(a):
            b = plsc.bitcast(a, jnp.int32)
            m = lax.shift_right_arithmetic(b, c31)
            return lax.bitwise_xor(b, lax.bitwise_or(m, minint))

        def wcompute(sel):
            sa0 = wacc[pl.ds(0, L)]
            sq0 = wacc[pl.ds(L, L)]

            @pl.loop(0, chunk // L, init_carry=(sa0, sq0))
            def acc(i, c):
                sa, sq = c
                a = buf[pl.ds(sel * chunk + i * L, L)]
                return (sa + jnp.abs(a), sq + a * a)

            sa, sq = acc
            wacc[pl.ds(0, L)] = sa
            wacc[pl.ds(L, L)] = sq

        wacc[pl.ds(0, L)] = jnp.zeros((L,), jnp.float32)
        wacc[pl.ds(L, L)] = jnp.zeros((L,), jnp.float32)
        stream(w_hbm, wid * wper, wch, wcompute)
        plsc.subcore_barrier()
        pltpu.sync_copy(wacc, wstage_sh.at[pl.ds(wid * 32, 32)])
        plsc.subcore_barrier()

        def select(row_base, thresh):
            def bod(j, c):
                r, jstar, rbef = c
                s = jnp.sum(gh[pl.ds(row_base + j * L, L)])
                newr = r + s
                crossed = jnp.logical_and(r < thresh, newr >= thresh)
                jstar = jnp.where(crossed, j, jstar)
                rbef = jnp.where(crossed, r, rbef)
                return (newr, jstar, rbef)

            _, jstar, rbef = lax.fori_loop(
                0, 16, bod,
                (jnp.int32(0), jnp.int32(0), jnp.int32(0)))
            v = gh[pl.ds(row_base + jstar * L, L)]
            cum = plsc.cumsum(v) + rbef
            off = _scalarize(plsc.all_reduce_ffs(cum >= thresh))
            byte = jstar * L + off
            hv = jnp.sum(jnp.where(iota == off, v, 0))
            cumat = jnp.sum(jnp.where(iota == off, cum, 0))
            return byte, cumat - hv

        def radix_pass(p, plo, phi, cblo, cbhi):
            sh_byte = jnp.full((L,), 24 - 8 * p, jnp.int32)
            sh_hi = jnp.full((L,), max(32 - 8 * p, 0), jnp.int32)
            lane_base = iota * 256

            @pl.loop(0, 8192 // L)
            def _(i):
                hist[pl.ds(i * L, L)] = jnp.zeros((L,), jnp.int32)

            def compute_fn(sel):
                @pl.loop(0, chunk // L)
                def _(i):
                    a = buf[pl.ds(sel * chunk + i * L, L)]
                    key = keys_of(a)
                    byte = lax.bitwise_and(
                        lax.shift_right_logical(key, sh_byte), c255)
                    idx = byte + lane_base
                    if p == 0:
                        plsc.addupdate_scatter(hist, [idx], one_i)
                    else:
                        hi = lax.shift_right_logical(key, sh_hi)
                        plsc.addupdate_scatter(
                            hist, [idx], one_i, mask=(hi == plo))
                        plsc.addupdate_scatter(
                            hist, [idx + 4096], one_i, mask=(hi == phi))

            stream(x_hbm, wid * per_w, nch, compute_fn)

            ntarget = 1 if p == 0 else 2

            @pl.loop(0, ntarget * 16)
            def _(tj):
                t = tj // 16
                j = lax.rem(tj, 16)

                @pl.loop(0, 16, init_carry=jnp.zeros((L,), jnp.int32))
                def acc(l, c):
                    return c + hist[pl.ds(t * 4096 + l * 256 + j * L, L)]

                folded[pl.ds(t * 256 + j * L, L)] = acc

            plsc.subcore_barrier()
            pltpu.sync_copy(folded, stage_sh.at[pl.ds(wid * 512, 512)])
            plsc.subcore_barrier()

            @pl.loop(0, 512 // L)
            def _(i):
                gh[pl.ds(i * L, L)] = jnp.zeros((L,), jnp.int32)

            @pl.loop(0, nw)
            def _(w):
                pltpu.sync_copy(stage_sh.at[pl.ds(w * 512, 512)], tmp)

                @pl.loop(0, 512 // L)
                def _(i):
                    gh[pl.ds(i * L, L)] = (
                        gh[pl.ds(i * L, L)] + tmp[pl.ds(i * L, L)])

            byte_lo, below_lo = select(0, k_lo - cblo)
            byte_hi, below_hi = select(0 if p == 0 else 256, k_hi - cbhi)
            plo = lax.bitwise_or(lax.shift_left(plo, jnp.int32(8)), byte_lo)
            phi = lax.bitwise_or(lax.shift_left(phi, jnp.int32(8)), byte_hi)
            return plo, phi, cblo + below_lo, cbhi + below_hi

        plo = phi = cblo = cbhi = jnp.int32(0)
        for p in range(4):
            plo, phi, cblo, cbhi = radix_pass(p, plo, phi, cblo, cbhi)

        z16 = jnp.zeros((L,), jnp.float32)

        @pl.loop(0, nw, init_carry=(z16, z16))
        def wtot(w, c):
            sa, sq = c
            pltpu.sync_copy(wstage_sh.at[pl.ds(w * 32, 32)], wtmp)
            return (sa + wtmp[pl.ds(0, L)], sq + wtmp[pl.ds(L, L)])

        sab = jnp.sum(wtot[0])
        ssq = jnp.sum(wtot[1])

        kv = jnp.where(iota == 0, phi, jnp.where(iota == 1, plo, 0))
        s = lax.shift_right_arithmetic(kv, c31)
        msk = lax.bitwise_xor(
            minint, lax.bitwise_and(maxint, lax.bitwise_not(s)))
        vals = plsc.bitcast(lax.bitwise_xor(kv, msk), jnp.float32)
        res = jnp.where(iota <= 1, vals, 0.0)
        res = jnp.where(iota == 2, sab, res)
        res = jnp.where(iota == 3, ssq, res)
        outv[...] = res

        @pl.when(wid == 0)
        def _():
            pltpu.sync_copy(outv, out_hbm)

    return body, out_type, scratch_types


@functools.lru_cache(maxsize=None)
def _build(n, wn):
    nw = 16
    chunk = 32768
    per_low = 0.1 * 0.01
    per_high = 99.9 * 0.01
    k_lo = max(int(per_low * n), 1)
    k_hi = int(per_high * n)
    body, out_type, scratch_types = _make_body(n, wn, nw, chunk, k_lo, k_hi)
    mesh = plsc.VectorSubcoreMesh(
        core_axis_name="c", subcore_axis_name="s", num_cores=1)
    return pl.kernel(body, out_type=out_type, mesh=mesh,
                     scratch_types=scratch_types,
                     compiler_params=pltpu.CompilerParams(
                         needs_layout_passes=False))


def kernel(x, weight):
    xf = x.reshape(-1)
    wf = weight.reshape(-1)
    fn = _build(xf.shape[0], wf.shape[0])
    res = fn(xf, wf)
    upper = res[0]
    lower = res[1]
    w_abs_mean = res[2] / wf.shape[0]
    w_std_sawb = jnp.sqrt(res[3] / wf.shape[0])
    w_clip = -12.8 * w_abs_mean + 12.68 * w_std_sawb
    return jnp.stack([upper, lower, w_clip])
